# Initial kernel scaffold; baseline (speedup 1.0000x reference)
import jax
import jax.numpy as jnp
from jax.experimental import pallas as pl


def kernel(x_user, x_item, edge_index_user_to_item, edge_index_item_to_user, enc_user_w, enc_user_b, enc_item_w, enc_item_b, u2i_wl0, u2i_bl0, u2i_wr0, i2u_wl0, i2u_bl0, i2u_wr0, u2i_wl1, u2i_bl1, u2i_wr1, i2u_wl1, i2u_bl1, i2u_wr1, head_w1, head_b1, head_w2, head_b2):
    raise NotImplementedError("write your pallas kernel here")


# speedup vs baseline: 5.7298x; 5.7298x over previous
import functools

import jax
import jax.numpy as jnp
from jax import lax
from jax.experimental import pallas as pl
from jax.experimental.pallas import tpu as pltpu
from jax.experimental.pallas import tpu_sc as plsc

NU, NI, E, D, H, OUTD = 10000, 10000, 320000, 128, 128, 16
NC, NS = 2, 16
NW = NC * NS
EW = E // NW
CHUNK = 80
NCHUNK = EW // CHUNK
RPT = 624
TAIL0 = NS * RPT


def _agg_body(h_hbm, src_hbm, dst_hbm, out_sum,
              src_a, dst_a, src_b, dst_b, rows_a, rows_b, ones_v, acc_sh,
              sem_ga, sem_gb, sem_sa, sem_sb, *, gather):
    c = lax.axis_index("c")
    s = lax.axis_index("s")
    wid = s * NC + c
    r0 = pl.multiple_of(s * RPT, 8)
    tb = pl.multiple_of(TAIL0 + (s % 2) * 8, 8)

    z16 = jnp.zeros((16,), jnp.float32)
    one16 = jnp.ones((16,), jnp.float32)

    def fill(i, carry):
        for j in range(H // 16):
            rows_a[i, pl.ds(j * 16, 16)] = z16
            if not gather:
                ones_v[i, pl.ds(j * 16, 16)] = one16
        return carry
    lax.fori_loop(0, CHUNK, fill, None)

    for t in range(7):
        pltpu.sync_copy(rows_a, acc_sh.at[pl.ds(r0 + t * CHUNK, CHUNK)])
    pltpu.sync_copy(rows_a.at[pl.ds(0, 64)], acc_sh.at[pl.ds(r0 + 560, 64)])
    pltpu.sync_copy(rows_a.at[pl.ds(0, 8)], acc_sh.at[pl.ds(tb, 8)])
    plsc.subcore_barrier()

    def load_idx(k, sv, dv):
        base = pl.multiple_of(wid * EW + k * CHUNK, 8)
        if gather:
            pltpu.sync_copy(src_hbm.at[pl.ds(base, CHUNK)], sv)
        pltpu.sync_copy(dst_hbm.at[pl.ds(base, CHUNK)], dv)

    ga_start = lambda: pltpu.async_copy(h_hbm.at[src_a], rows_a, sem_ga)
    gb_start = lambda: pltpu.async_copy(h_hbm.at[src_b], rows_b, sem_gb)
    ga_wait = lambda: pltpu.make_async_copy(h_hbm.at[src_a], rows_a, sem_ga).wait()
    gb_wait = lambda: pltpu.make_async_copy(h_hbm.at[src_b], rows_b, sem_gb).wait()
    upd_a = rows_a if gather else ones_v
    upd_b = rows_b if gather else ones_v
    sa_start = lambda: pltpu.async_copy(upd_a, acc_sh.at[dst_a], sem_sa, add=True)
    sb_start = lambda: pltpu.async_copy(upd_b, acc_sh.at[dst_b], sem_sb, add=True)
    sa_wait = lambda: pltpu.make_async_copy(upd_a, acc_sh.at[dst_a], sem_sa).wait()
    sb_wait = lambda: pltpu.make_async_copy(upd_b, acc_sh.at[dst_b], sem_sb).wait()

    load_idx(0, src_a, dst_a)
    if gather:
        ga_start()
        ga_wait()
    sa_start()
    load_idx(1, src_b, dst_b)
    if gather:
        gb_start()
    sa_wait()
    load_idx(2, src_a, dst_a)
    if gather:
        ga_start()
        gb_wait()
    sb_start()

    def pair(g, carry):
        k = 2 * g
        if gather:
            ga_wait()
        sa_start()
        sb_wait()
        load_idx(k + 1, src_b, dst_b)
        if gather:
            gb_start()
        sa_wait()
        load_idx(k + 2, src_a, dst_a)
        if gather:
            ga_start()
            gb_wait()
        sb_start()
        return carry
    lax.fori_loop(1, 62, pair, None)

    if gather:
        ga_wait()
    sa_start()
    sb_wait()
    sa_wait()
    plsc.subcore_barrier()

    for t in range(7):
        o = pl.multiple_of(r0 + t * CHUNK, 8)
        pltpu.sync_copy(acc_sh.at[pl.ds(o, CHUNK)], rows_a)
        pltpu.sync_copy(rows_a, out_sum.at[c, pl.ds(o, CHUNK)])
    o = pl.multiple_of(r0 + 560, 8)
    pltpu.sync_copy(acc_sh.at[pl.ds(o, 64)], rows_a.at[pl.ds(0, 64)])
    pltpu.sync_copy(rows_a.at[pl.ds(0, 64)], out_sum.at[c, pl.ds(o, 64)])
    pltpu.sync_copy(acc_sh.at[pl.ds(tb, 8)], rows_a.at[pl.ds(0, 8)])
    pltpu.sync_copy(rows_a.at[pl.ds(0, 8)], out_sum.at[c, pl.ds(tb, 8)])


def _make_agg(gather):
    mesh = plsc.VectorSubcoreMesh(core_axis_name="c", subcore_axis_name="s")
    return pl.kernel(
        functools.partial(_agg_body, gather=gather),
        mesh=mesh,
        out_type=jax.ShapeDtypeStruct((NC, NU, H), jnp.float32),
        scratch_types=[
            pltpu.VMEM((CHUNK,), jnp.int32),
            pltpu.VMEM((CHUNK,), jnp.int32),
            pltpu.VMEM((CHUNK,), jnp.int32),
            pltpu.VMEM((CHUNK,), jnp.int32),
            pltpu.VMEM((CHUNK, H), jnp.float32),
            pltpu.VMEM((CHUNK, H), jnp.float32),
            pltpu.VMEM((CHUNK, H), jnp.float32),
            pltpu.VMEM_SHARED((NU, H), jnp.float32),
            pltpu.SemaphoreType.DMA,
            pltpu.SemaphoreType.DMA,
            pltpu.SemaphoreType.DMA,
            pltpu.SemaphoreType.DMA,
        ],
    )


_RB = 1000


def _enc_kernel(x_ref, w_ref, b_ref, o_ref):
    o_ref[...] = jnp.maximum(
        jnp.dot(x_ref[...], w_ref[...], preferred_element_type=jnp.float32)
        + b_ref[...], 0.0)


def _encode(x, w, b):
    n = x.shape[0]
    return pl.pallas_call(
        _enc_kernel,
        grid=(n // _RB,),
        in_specs=[
            pl.BlockSpec((_RB, D), lambda i: (i, 0)),
            pl.BlockSpec((D, H), lambda i: (0, 0)),
            pl.BlockSpec((1, H), lambda i: (0, 0)),
        ],
        out_specs=pl.BlockSpec((_RB, H), lambda i: (i, 0)),
        out_shape=jax.ShapeDtypeStruct((n, H), jnp.float32),
    )(x, w, b.reshape(1, H))


def _sage_kernel(sum_ref, cnt_ref, hdst_ref, wl_ref, bl_ref, wr_ref, o_ref):
    sblk = sum_ref[0] + sum_ref[1]
    cblk = cnt_ref[0, :, 0:1] + cnt_ref[1, :, 0:1]
    mean = sblk / jnp.maximum(cblk, 1.0)
    hdst = hdst_ref[...]
    out = (jnp.dot(mean, wl_ref[...], preferred_element_type=jnp.float32)
           + bl_ref[...]
           + jnp.dot(hdst, wr_ref[...], preferred_element_type=jnp.float32))
    nrm = jnp.sqrt(jnp.sum(out * out, axis=-1, keepdims=True))
    out = out / jnp.maximum(nrm, 1e-12)
    o_ref[...] = jnp.maximum(out, 0.0) + hdst


def _sage_finish(sums, cnts, h_dst, wl, bl, wr):
    n = h_dst.shape[0]
    return pl.pallas_call(
        _sage_kernel,
        grid=(n // _RB,),
        in_specs=[
            pl.BlockSpec((NC, _RB, H), lambda i: (0, i, 0)),
            pl.BlockSpec((NC, _RB, H), lambda i: (0, i, 0)),
            pl.BlockSpec((_RB, H), lambda i: (i, 0)),
            pl.BlockSpec((H, H), lambda i: (0, 0)),
            pl.BlockSpec((1, H), lambda i: (0, 0)),
            pl.BlockSpec((H, H), lambda i: (0, 0)),
        ],
        out_specs=pl.BlockSpec((_RB, H), lambda i: (i, 0)),
        out_shape=jax.ShapeDtypeStruct((n, H), jnp.float32),
    )(sums, cnts, h_dst, wl, bl.reshape(1, H), wr)


def _head_kernel(x_ref, w1_ref, b1_ref, w2_ref, b2_ref, o_ref):
    z = jnp.maximum(
        jnp.dot(x_ref[...], w1_ref[...], preferred_element_type=jnp.float32)
        + b1_ref[...], 0.0)
    o_ref[...] = (jnp.dot(z, w2_ref[...], preferred_element_type=jnp.float32)
                  + b2_ref[...])


def _head(x, w1, b1, w2, b2):
    n = x.shape[0]
    hh = w1.shape[1]
    return pl.pallas_call(
        _head_kernel,
        grid=(n // _RB,),
        in_specs=[
            pl.BlockSpec((_RB, H), lambda i: (i, 0)),
            pl.BlockSpec((H, hh), lambda i: (0, 0)),
            pl.BlockSpec((1, hh), lambda i: (0, 0)),
            pl.BlockSpec((hh, OUTD), lambda i: (0, 0)),
            pl.BlockSpec((1, OUTD), lambda i: (0, 0)),
        ],
        out_specs=pl.BlockSpec((_RB, OUTD), lambda i: (i, 0)),
        out_shape=jax.ShapeDtypeStruct((n, OUTD), jnp.float32),
    )(x, w1, b1.reshape(1, hh), w2, b2.reshape(1, OUTD))


def kernel(x_user, x_item, edge_index_user_to_item, edge_index_item_to_user,
           enc_user_w, enc_user_b, enc_item_w, enc_item_b,
           u2i_wl0, u2i_bl0, u2i_wr0, i2u_wl0, i2u_bl0, i2u_wr0,
           u2i_wl1, u2i_bl1, u2i_wr1, i2u_wl1, i2u_bl1, i2u_wr1,
           head_w1, head_b1, head_w2, head_b2):
    agg = _make_agg(gather=True)
    deg = _make_agg(gather=False)

    src_u2i = edge_index_user_to_item[0]
    dst_u2i = edge_index_user_to_item[1]
    src_i2u = edge_index_item_to_user[0]
    dst_i2u = edge_index_item_to_user[1]

    hu = _encode(x_user, enc_user_w, enc_user_b)
    hi = _encode(x_item, enc_item_w, enc_item_b)

    cnt_i = deg(hu, src_u2i, dst_u2i)
    cnt_u = deg(hi, src_i2u, dst_i2u)

    layer_w = (
        (u2i_wl0, u2i_bl0, u2i_wr0, i2u_wl0, i2u_bl0, i2u_wr0),
        (u2i_wl1, u2i_bl1, u2i_wr1, i2u_wl1, i2u_bl1, i2u_wr1),
    )
    for (wl_i, bl_i, wr_i, wl_u, bl_u, wr_u) in layer_w:
        sum_i = agg(hu, src_u2i, dst_u2i)
        sum_u = agg(hi, src_i2u, dst_i2u)
        hi_new = _sage_finish(sum_i, cnt_i, hi, wl_i, bl_i, wr_i)
        hu_new = _sage_finish(sum_u, cnt_u, hu, wl_u, bl_u, wr_u)
        hu, hi = hu_new, hi_new

    return _head(hu, head_w1, head_b1, head_w2, head_b2)
